# baseline (speedup 1.0000x reference)
import functools

import jax
import jax.numpy as jnp
from jax import lax
from jax.experimental import pallas as pl
from jax.experimental.pallas import tpu as pltpu
from jax.experimental.pallas import tpu_sc as plsc

_B = 16384
_D = 64
_NC = 2
_NS = 16
_NW = _NC * _NS
_BPW = _B // _NW
_CH = 4
_CB = _BPW // _CH
_L = 16


def _body(hidx_hbm, ridx_hbm, tidx_hbm, ent_hbm, rel_hbm, wb_hbm, out_hbm,
          hidx_v, ridx_v, tidx_v, h_v, r_v, t_v, out_v, tr_v, wb_v,
          sem_h, sem_r, sem_t):
    wid = lax.axis_index("s") * _NC + lax.axis_index("c")
    row0 = wid * _CH

    pltpu.sync_copy(hidx_hbm.at[pl.ds(row0, _CH)], hidx_v)
    pltpu.sync_copy(ridx_hbm.at[pl.ds(row0, _CH)], ridx_v)
    pltpu.sync_copy(tidx_hbm.at[pl.ds(row0, _CH)], tidx_v)
    pltpu.sync_copy(wb_hbm, wb_v)

    copies = []
    for c in range(_CH):
        copies.append(pltpu.async_copy(ent_hbm.at[hidx_v.at[c]], h_v.at[c], sem_h))
        copies.append(pltpu.async_copy(rel_hbm.at[ridx_v.at[c]], r_v.at[c], sem_r))
        copies.append(pltpu.async_copy(ent_hbm.at[tidx_v.at[c]], t_v.at[c], sem_t))
    for cp in copies:
        cp.wait()

    iota = lax.iota(jnp.int32, _L)
    col_idx = iota * _L
    wvec = wb_v[0, :]
    bvec = wb_v[1, :]

    for c in range(_CH):
        hc, rc, tc = h_v.at[c], r_v.at[c], t_v.at[c]

        def group(g, _, c=c, hc=hc, rc=rc, tc=tc):
            for j in range(_L):
                row = g * _L + j
                s = (hc[row, pl.ds(0, _L)] * rc[row, pl.ds(0, _L)]
                     * tc[row, pl.ds(0, _L)])
                for d in range(1, _D // _L):
                    s = s + (hc[row, pl.ds(d * _L, _L)]
                             * rc[row, pl.ds(d * _L, _L)]
                             * tc[row, pl.ds(d * _L, _L)])
                tr_v[pl.ds(j * _L, _L)] = s
            acc = plsc.load_gather(tr_v, [col_idx])
            for d in range(1, _L):
                acc = acc + plsc.load_gather(tr_v, [col_idx + d])
            x = wvec * acc + bvec
            score = 1.0 / (1.0 + jnp.exp(-x))
            out_v[pl.ds((c * (_CB // _L) + g) * _L, _L)] = score
            return _

        lax.fori_loop(0, _CB // _L, group, 0, unroll=False)

    pltpu.sync_copy(out_v, out_hbm.at[pl.ds(wid * _BPW, _BPW)])


@jax.jit
def _run(hidx, ridx, tidx, ent_emb, rel_emb, wb):
    mesh = plsc.VectorSubcoreMesh(core_axis_name="c", subcore_axis_name="s",
                                  num_cores=_NC, num_subcores=_NS)
    return pl.kernel(
        _body,
        out_type=jax.ShapeDtypeStruct((_B,), jnp.float32),
        mesh=mesh,
        scratch_types=[
            pltpu.VMEM((_CH, _CB), jnp.int32),
            pltpu.VMEM((_CH, _CB), jnp.int32),
            pltpu.VMEM((_CH, _CB), jnp.int32),
            pltpu.VMEM((_CH, _CB, _D), jnp.float32),
            pltpu.VMEM((_CH, _CB, _D), jnp.float32),
            pltpu.VMEM((_CH, _CB, _D), jnp.float32),
            pltpu.VMEM((_BPW,), jnp.float32),
            pltpu.VMEM((_L * _L,), jnp.float32),
            pltpu.VMEM((2, _L), jnp.float32),
            pltpu.SemaphoreType.DMA,
            pltpu.SemaphoreType.DMA,
            pltpu.SemaphoreType.DMA,
        ],
        compiler_params=pltpu.CompilerParams(needs_layout_passes=False,
                                             use_tc_tiling_on_sc=False),
    )(hidx, ridx, tidx, ent_emb, rel_emb, wb)


def kernel(triples, ent_emb, rel_emb, w, b):
    tri = triples.astype(jnp.int32)
    hidx = tri[:, 0].reshape(_NW * _CH, _CB)
    ridx = tri[:, 1].reshape(_NW * _CH, _CB)
    tidx = tri[:, 2].reshape(_NW * _CH, _CB)
    wb = jnp.stack([jnp.full((_L,), w, jnp.float32),
                    jnp.full((_L,), b, jnp.float32)])
    return _run(hidx, ridx, tidx, ent_emb, rel_emb, wb)
